# baseline (speedup 1.0000x reference)
import functools

import jax
import jax.numpy as jnp
from jax import lax
from jax.experimental import pallas as pl
from jax.experimental.pallas import tpu as pltpu
from jax.experimental.pallas import tpu_sc as plsc

N_NODES = 10000
N_EDGES = 320000
D = 128

NC = 2
NS = 16
NW = NC * NS
NPAD = 10240
PAD_NODE = NPAD - 1
ROWS_PER_TILE = NPAD // NS
EPW = N_EDGES // NW
K = 128
EPWP = 10240
CHUNKS = EPWP // K
ZR = 16


def _worker_id():
  c = lax.axis_index("c")
  s = lax.axis_index("s")
  return c, s, s * NC + c


def _zero_fill_vmem(zbuf, nwords):
  zeros16 = jnp.zeros((16,), jnp.float32)

  def body(j, _):
    zbuf[pl.ds(j * 16, 16)] = zeros16
    return 0

  lax.fori_loop(0, nwords // 16, body, 0)


def _zero_fill_vmem_2d(zbuf, rows, cols):
  zeros16 = jnp.zeros((16,), jnp.float32)
  cpr = cols // 16

  def body(t, _):
    r = t // cpr
    c = lax.rem(t, cpr)
    zbuf[r, pl.ds(c * 16, 16)] = zeros16
    return 0

  lax.fori_loop(0, rows * cpr, body, 0)


def _sc_deg_body(rc_hbm, out_hbm, onesv, rcv, zbuf, deg_sh, sem):
  c, s, wid = _worker_id()

  _zero_fill_vmem(zbuf, ROWS_PER_TILE)
  pltpu.sync_copy(zbuf, deg_sh.at[pl.ds(s * ROWS_PER_TILE, ROWS_PER_TILE)])

  def fill_ones(j, _):
    onesv[pl.ds(j * 16, 16)] = jnp.ones((16,), jnp.float32)
    return 0

  lax.fori_loop(0, K // 16, fill_ones, 0)
  pltpu.sync_copy(rc_hbm.at[wid], rcv)
  plsc.subcore_barrier()

  def chunk(i, _):
    pltpu.sync_copy(onesv, deg_sh.at[rcv.at[i, 1]], add=True)
    return 0

  lax.fori_loop(0, CHUNKS, chunk, 0)
  plsc.subcore_barrier()

  pltpu.sync_copy(
      deg_sh.at[pl.ds(s * ROWS_PER_TILE, ROWS_PER_TILE)],
      out_hbm.at[c, pl.ds(s * ROWS_PER_TILE, ROWS_PER_TILE)],
  )


def _make_sc_deg():
  mesh = plsc.VectorSubcoreMesh(
      core_axis_name="c", subcore_axis_name="s", num_cores=NC, num_subcores=NS
  )
  return pl.kernel(
      _sc_deg_body,
      out_type=jax.ShapeDtypeStruct((NC, NPAD), jnp.float32),
      mesh=mesh,
      scratch_types=[
          pltpu.VMEM((K,), jnp.float32),
          pltpu.VMEM((CHUNKS, 2, K), jnp.int32),
          pltpu.VMEM((ROWS_PER_TILE,), jnp.float32),
          pltpu.VMEM_SHARED((NPAD,), jnp.float32),
          pltpu.SemaphoreType.DMA,
      ],
  )


def _sc_agg_body(rc_hbm, y_hbm, out_hbm, rcv0, rcv1, buf0, buf1,
                 zbuf, z_sh, sem0, sem1):
  c, s, wid = _worker_id()

  _zero_fill_vmem_2d(zbuf, ZR, D)

  def zslab(j, _):
    pltpu.sync_copy(zbuf, z_sh.at[pl.ds(s * ROWS_PER_TILE + j * ZR, ZR)])
    return 0

  lax.fori_loop(0, ROWS_PER_TILE // ZR, zslab, 0)
  plsc.subcore_barrier()

  pltpu.sync_copy(rc_hbm.at[wid, 0], rcv0)
  pltpu.async_copy(y_hbm.at[rcv0.at[0]], buf0, sem0)
  npairs = CHUNKS // 2

  def pair(j, _):
    e0 = 2 * j
    pltpu.sync_copy(rc_hbm.at[wid, e0 + 1], rcv1)
    pltpu.make_async_copy(y_hbm.at[rcv0.at[0]], buf0, sem0).wait()
    pltpu.async_copy(y_hbm.at[rcv1.at[0]], buf1, sem1)
    pltpu.sync_copy(buf0, z_sh.at[rcv0.at[1]], add=True)

    @pl.when(j < npairs - 1)
    def _():
      pltpu.sync_copy(rc_hbm.at[wid, e0 + 2], rcv0)

    pltpu.make_async_copy(y_hbm.at[rcv1.at[0]], buf1, sem1).wait()

    @pl.when(j < npairs - 1)
    def _():
      pltpu.async_copy(y_hbm.at[rcv0.at[0]], buf0, sem0)

    pltpu.sync_copy(buf1, z_sh.at[rcv1.at[1]], add=True)
    return 0

  lax.fori_loop(0, npairs, pair, 0)
  plsc.subcore_barrier()

  pltpu.sync_copy(
      z_sh.at[pl.ds(s * ROWS_PER_TILE, ROWS_PER_TILE)],
      out_hbm.at[c, pl.ds(s * ROWS_PER_TILE, ROWS_PER_TILE)],
  )


def _make_sc_agg():
  mesh = plsc.VectorSubcoreMesh(
      core_axis_name="c", subcore_axis_name="s", num_cores=NC, num_subcores=NS
  )
  return pl.kernel(
      _sc_agg_body,
      out_type=jax.ShapeDtypeStruct((NC, NPAD, D), jnp.float32),
      mesh=mesh,
      scratch_types=[
          pltpu.VMEM((2, K), jnp.int32),
          pltpu.VMEM((2, K), jnp.int32),
          pltpu.VMEM((K, D), jnp.float32),
          pltpu.VMEM((K, D), jnp.float32),
          pltpu.VMEM((ZR, D), jnp.float32),
          pltpu.VMEM_SHARED((NPAD, D), jnp.float32),
          pltpu.SemaphoreType.DMA,
          pltpu.SemaphoreType.DMA,
      ],
  )


BR = 256


def _tc_first_body(x_ref, w_ref, d0_ref, d1_ref, ys_ref, dis_ref):
  dis = lax.rsqrt(d0_ref[...] + d1_ref[...] + 1.0)
  y = jnp.dot(x_ref[...], w_ref[...], preferred_element_type=jnp.float32)
  ys_ref[...] = y * dis
  dis_ref[...] = dis


def _tc_first(x, w, d0, d1):
  grid = (NPAD // BR,)
  return pl.pallas_call(
      _tc_first_body,
      grid=grid,
      in_specs=[
          pl.BlockSpec((BR, D), lambda i: (i, 0)),
          pl.BlockSpec((D, D), lambda i: (0, 0)),
          pl.BlockSpec((BR, 1), lambda i: (i, 0)),
          pl.BlockSpec((BR, 1), lambda i: (i, 0)),
      ],
      out_specs=[
          pl.BlockSpec((BR, D), lambda i: (i, 0)),
          pl.BlockSpec((BR, 1), lambda i: (i, 0)),
      ],
      out_shape=[
          jax.ShapeDtypeStruct((NPAD, D), jnp.float32),
          jax.ShapeDtypeStruct((NPAD, 1), jnp.float32),
      ],
  )(x, w, d0, d1)


def _tc_mid_body(p0_ref, p1_ref, ys_ref, dis_ref, b_ref, w_ref, out_ref):
  dis = dis_ref[...]
  z = (p0_ref[...] + p1_ref[...] + ys_ref[...]) * dis + b_ref[...]
  h = jnp.maximum(z, 0.0)
  out_ref[...] = (
      jnp.dot(h, w_ref[...], preferred_element_type=jnp.float32) * dis
  )


def _tc_mid(p0, p1, ys, dis, b, w):
  grid = (NPAD // BR,)
  return pl.pallas_call(
      _tc_mid_body,
      grid=grid,
      in_specs=[
          pl.BlockSpec((BR, D), lambda i: (i, 0)),
          pl.BlockSpec((BR, D), lambda i: (i, 0)),
          pl.BlockSpec((BR, D), lambda i: (i, 0)),
          pl.BlockSpec((BR, 1), lambda i: (i, 0)),
          pl.BlockSpec((1, D), lambda i: (0, 0)),
          pl.BlockSpec((D, D), lambda i: (0, 0)),
      ],
      out_specs=pl.BlockSpec((BR, D), lambda i: (i, 0)),
      out_shape=jax.ShapeDtypeStruct((NPAD, D), jnp.float32),
  )(p0, p1, ys, dis, b, w)


def _tc_last_body(q0_ref, q1_ref, ys_ref, dis_ref, b_ref, out_ref):
  out_ref[...] = (
      (q0_ref[...] + q1_ref[...] + ys_ref[...]) * dis_ref[...] + b_ref[...]
  )


def _tc_last(q0, q1, ys, dis, b):
  grid = (NPAD // BR,)
  return pl.pallas_call(
      _tc_last_body,
      grid=grid,
      in_specs=[
          pl.BlockSpec((BR, D), lambda i: (i, 0)),
          pl.BlockSpec((BR, D), lambda i: (i, 0)),
          pl.BlockSpec((BR, D), lambda i: (i, 0)),
          pl.BlockSpec((BR, 1), lambda i: (i, 0)),
          pl.BlockSpec((1, D), lambda i: (0, 0)),
      ],
      out_specs=pl.BlockSpec((BR, D), lambda i: (i, 0)),
      out_shape=jax.ShapeDtypeStruct((NPAD, D), jnp.float32),
  )(q0, q1, ys, dis, b)


@jax.jit
def _run(label_embedding, edge_index, W1, b1, W2, b2):
  ei = edge_index.reshape(2, NW, EPW)
  ei = jnp.pad(ei, ((0, 0), (0, 0), (0, EPWP - EPW)),
               constant_values=PAD_NODE)
  rc = jnp.transpose(ei, (1, 0, 2)).reshape(NW, 2, CHUNKS, K)
  rc = jnp.transpose(rc, (0, 2, 1, 3))

  x = jnp.zeros((NPAD, D), jnp.float32).at[:N_NODES].set(label_embedding)
  b1r = b1.reshape(1, D)
  b2r = b2.reshape(1, D)

  sc_deg = _make_sc_deg()
  sc_agg = _make_sc_agg()

  degp = sc_deg(rc)
  d0 = degp[0].reshape(NPAD, 1)
  d1 = degp[1].reshape(NPAD, 1)

  ys1, dis = _tc_first(x, W1, d0, d1)
  p = sc_agg(rc, ys1)
  ys2 = _tc_mid(p[0], p[1], ys1, dis, b1r, W2)
  q = sc_agg(rc, ys2)
  out = _tc_last(q[0], q[1], ys2, dis, b2r)
  return out[:N_NODES]


def kernel(label_embedding, edge_index, W1, b1, W2, b2):
  return _run(label_embedding, edge_index, W1, b1, W2, b2)


# speedup vs baseline: 2.1144x; 2.1144x over previous
import functools

import jax
import jax.numpy as jnp
from jax import lax
from jax.experimental import pallas as pl
from jax.experimental.pallas import tpu as pltpu
from jax.experimental.pallas import tpu_sc as plsc

N_NODES = 10000
N_EDGES = 320000
D = 128

NC = 2
NS = 16
NW = NC * NS
NPAD = 10240
ROWS_PER_TILE = NPAD // NS
EPW = N_EDGES // NW
K = 80
CHUNKS = EPW // K
ZR = 64


def _worker_id():
  c = lax.axis_index("c")
  s = lax.axis_index("s")
  return c, s, s * NC + c


def _zero_fill_vmem(zbuf, nwords):
  zeros16 = jnp.zeros((16,), jnp.float32)

  def body(j, _):
    zbuf[pl.ds(j * 16, 16)] = zeros16
    return 0

  lax.fori_loop(0, nwords // 16, body, 0)


def _zero_fill_vmem_2d(zbuf, rows, cols):
  zeros16 = jnp.zeros((16,), jnp.float32)
  cpr = cols // 16

  def body(t, _):
    r = t // cpr
    c = lax.rem(t, cpr)
    zbuf[r, pl.ds(c * 16, 16)] = zeros16
    return 0

  lax.fori_loop(0, rows * cpr, body, 0)


def _sc_deg_body(rc_hbm, out_hbm, onesv, rcv, zbuf, deg_sh, sem):
  c, s, wid = _worker_id()

  _zero_fill_vmem(zbuf, ROWS_PER_TILE)
  pltpu.sync_copy(zbuf, deg_sh.at[pl.ds(s * ROWS_PER_TILE, ROWS_PER_TILE)])

  def fill_ones(j, _):
    onesv[pl.ds(j * 16, 16)] = jnp.ones((16,), jnp.float32)
    return 0

  lax.fori_loop(0, K // 16, fill_ones, 0)
  pltpu.sync_copy(rc_hbm.at[wid], rcv)
  plsc.subcore_barrier()

  def chunk(i, _):
    pltpu.sync_copy(onesv, deg_sh.at[rcv.at[i, 1]], add=True)
    return 0

  lax.fori_loop(0, CHUNKS, chunk, 0)
  plsc.subcore_barrier()

  pltpu.sync_copy(
      deg_sh.at[pl.ds(s * ROWS_PER_TILE, ROWS_PER_TILE)],
      out_hbm.at[c, pl.ds(s * ROWS_PER_TILE, ROWS_PER_TILE)],
  )


def _make_sc_deg():
  mesh = plsc.VectorSubcoreMesh(
      core_axis_name="c", subcore_axis_name="s", num_cores=NC, num_subcores=NS
  )
  return pl.kernel(
      _sc_deg_body,
      out_type=jax.ShapeDtypeStruct((NC, NPAD), jnp.float32),
      mesh=mesh,
      scratch_types=[
          pltpu.VMEM((K,), jnp.float32),
          pltpu.VMEM((CHUNKS, 2, K), jnp.int32),
          pltpu.VMEM((ROWS_PER_TILE,), jnp.float32),
          pltpu.VMEM_SHARED((NPAD,), jnp.float32),
          pltpu.SemaphoreType.DMA,
      ],
  )


def _sc_agg_body(rc_hbm, y_hbm, out_hbm, rcv0, rcv1, buf0, buf1,
                 zbuf, z_sh, sem0, sem1):
  c, s, wid = _worker_id()

  _zero_fill_vmem_2d(zbuf, ZR, D)

  def zslab(j, _):
    pltpu.sync_copy(zbuf, z_sh.at[pl.ds(s * ROWS_PER_TILE + j * ZR, ZR)])
    return 0

  lax.fori_loop(0, ROWS_PER_TILE // ZR, zslab, 0)
  plsc.subcore_barrier()

  pltpu.sync_copy(rc_hbm.at[wid, 0], rcv0)
  pltpu.async_copy(y_hbm.at[rcv0.at[0]], buf0, sem0)
  npairs = CHUNKS // 2

  def pair(j, _):
    e0 = 2 * j
    pltpu.sync_copy(rc_hbm.at[wid, e0 + 1], rcv1)
    pltpu.make_async_copy(y_hbm.at[rcv0.at[0]], buf0, sem0).wait()
    pltpu.async_copy(y_hbm.at[rcv1.at[0]], buf1, sem1)
    pltpu.sync_copy(buf0, z_sh.at[rcv0.at[1]], add=True)
    pltpu.sync_copy(rc_hbm.at[wid, e0 + 2], rcv0)
    pltpu.make_async_copy(y_hbm.at[rcv1.at[0]], buf1, sem1).wait()
    pltpu.async_copy(y_hbm.at[rcv0.at[0]], buf0, sem0)
    pltpu.sync_copy(buf1, z_sh.at[rcv1.at[1]], add=True)
    return 0

  lax.fori_loop(0, npairs, pair, 0)
  pltpu.make_async_copy(y_hbm.at[rcv0.at[0]], buf0, sem0).wait()
  pltpu.sync_copy(buf0, z_sh.at[rcv0.at[1]], add=True)
  plsc.subcore_barrier()

  pltpu.sync_copy(
      z_sh.at[pl.ds(s * ROWS_PER_TILE, ROWS_PER_TILE)],
      out_hbm.at[c, pl.ds(s * ROWS_PER_TILE, ROWS_PER_TILE)],
  )


def _make_sc_agg():
  mesh = plsc.VectorSubcoreMesh(
      core_axis_name="c", subcore_axis_name="s", num_cores=NC, num_subcores=NS
  )
  return pl.kernel(
      _sc_agg_body,
      out_type=jax.ShapeDtypeStruct((NC, NPAD, D), jnp.float32),
      mesh=mesh,
      scratch_types=[
          pltpu.VMEM((2, K), jnp.int32),
          pltpu.VMEM((2, K), jnp.int32),
          pltpu.VMEM((K, D), jnp.float32),
          pltpu.VMEM((K, D), jnp.float32),
          pltpu.VMEM((ZR, D), jnp.float32),
          pltpu.VMEM_SHARED((NPAD, D), jnp.float32),
          pltpu.SemaphoreType.DMA,
          pltpu.SemaphoreType.DMA,
      ],
  )


BR = 256


def _tc_first_body(x_ref, w_ref, d0_ref, d1_ref, ys_ref, dis_ref):
  dis = lax.rsqrt(d0_ref[...] + d1_ref[...] + 1.0)
  y = jnp.dot(x_ref[...], w_ref[...], preferred_element_type=jnp.float32)
  ys_ref[...] = y * dis
  dis_ref[...] = dis


def _tc_first(x, w, d0, d1):
  grid = (NPAD // BR,)
  return pl.pallas_call(
      _tc_first_body,
      grid=grid,
      in_specs=[
          pl.BlockSpec((BR, D), lambda i: (i, 0)),
          pl.BlockSpec((D, D), lambda i: (0, 0)),
          pl.BlockSpec((BR, 1), lambda i: (i, 0)),
          pl.BlockSpec((BR, 1), lambda i: (i, 0)),
      ],
      out_specs=[
          pl.BlockSpec((BR, D), lambda i: (i, 0)),
          pl.BlockSpec((BR, 1), lambda i: (i, 0)),
      ],
      out_shape=[
          jax.ShapeDtypeStruct((NPAD, D), jnp.float32),
          jax.ShapeDtypeStruct((NPAD, 1), jnp.float32),
      ],
  )(x, w, d0, d1)


def _tc_mid_body(p0_ref, p1_ref, ys_ref, dis_ref, b_ref, w_ref, out_ref):
  dis = dis_ref[...]
  z = (p0_ref[...] + p1_ref[...] + ys_ref[...]) * dis + b_ref[...]
  h = jnp.maximum(z, 0.0)
  out_ref[...] = (
      jnp.dot(h, w_ref[...], preferred_element_type=jnp.float32) * dis
  )


def _tc_mid(p0, p1, ys, dis, b, w):
  grid = (NPAD // BR,)
  return pl.pallas_call(
      _tc_mid_body,
      grid=grid,
      in_specs=[
          pl.BlockSpec((BR, D), lambda i: (i, 0)),
          pl.BlockSpec((BR, D), lambda i: (i, 0)),
          pl.BlockSpec((BR, D), lambda i: (i, 0)),
          pl.BlockSpec((BR, 1), lambda i: (i, 0)),
          pl.BlockSpec((1, D), lambda i: (0, 0)),
          pl.BlockSpec((D, D), lambda i: (0, 0)),
      ],
      out_specs=pl.BlockSpec((BR, D), lambda i: (i, 0)),
      out_shape=jax.ShapeDtypeStruct((NPAD, D), jnp.float32),
  )(p0, p1, ys, dis, b, w)


def _tc_last_body(q0_ref, q1_ref, ys_ref, dis_ref, b_ref, out_ref):
  out_ref[...] = (
      (q0_ref[...] + q1_ref[...] + ys_ref[...]) * dis_ref[...] + b_ref[...]
  )


def _tc_last(q0, q1, ys, dis, b):
  grid = (NPAD // BR,)
  return pl.pallas_call(
      _tc_last_body,
      grid=grid,
      in_specs=[
          pl.BlockSpec((BR, D), lambda i: (i, 0)),
          pl.BlockSpec((BR, D), lambda i: (i, 0)),
          pl.BlockSpec((BR, D), lambda i: (i, 0)),
          pl.BlockSpec((BR, 1), lambda i: (i, 0)),
          pl.BlockSpec((1, D), lambda i: (0, 0)),
      ],
      out_specs=pl.BlockSpec((BR, D), lambda i: (i, 0)),
      out_shape=jax.ShapeDtypeStruct((NPAD, D), jnp.float32),
  )(q0, q1, ys, dis, b)


@jax.jit
def _run(label_embedding, edge_index, W1, b1, W2, b2):
  rc = edge_index.reshape(2, NW, CHUNKS, K)
  rc = jnp.transpose(rc, (1, 2, 0, 3))

  x = jnp.zeros((NPAD, D), jnp.float32).at[:N_NODES].set(label_embedding)
  b1r = b1.reshape(1, D)
  b2r = b2.reshape(1, D)

  sc_deg = _make_sc_deg()
  sc_agg = _make_sc_agg()

  degp = sc_deg(rc)
  d0 = degp[0].reshape(NPAD, 1)
  d1 = degp[1].reshape(NPAD, 1)

  ys1, dis = _tc_first(x, W1, d0, d1)
  p = sc_agg(rc, ys1)
  ys2 = _tc_mid(p[0], p[1], ys1, dis, b1r, W2)
  q = sc_agg(rc, ys2)
  out = _tc_last(q[0], q[1], ys2, dis, b2r)
  return out[:N_NODES]


def kernel(label_embedding, edge_index, W1, b1, W2, b2):
  return _run(label_embedding, edge_index, W1, b1, W2, b2)
